# baseline (speedup 1.0000x reference)
import functools

import jax
import jax.numpy as jnp
from jax import lax
from jax.experimental import pallas as pl
from jax.experimental.pallas import tpu as pltpu
from jax.experimental.pallas import tpu_sc as plsc

NC = 2
NS = 16
NW = NC * NS


def _tc_transform(table, w_down, w_up, bm=8000):
    v, d = table.shape
    h = w_down.shape[0]
    assert v % bm == 0

    def mm(t_hbm, wd_ref, wu_ref, out_ref, t_vmem, sem):
        i = pl.program_id(0)
        pltpu.make_async_copy(
            t_hbm.at[pl.ds(i * bm, bm)], t_vmem, sem).start()
        wc = lax.dot_general(
            wd_ref[...], wu_ref[...], (((0,), (1,)), ((), ())),
            preferred_element_type=jnp.float32)
        pltpu.make_async_copy(
            t_hbm.at[pl.ds(i * bm, bm)], t_vmem, sem).wait()
        res = lax.dot_general(
            t_vmem[...], wc, (((1,), (0,)), ((), ())),
            preferred_element_type=jnp.float32)
        out_ref[...] = jnp.concatenate([res, res], axis=1)

    return pl.pallas_call(
        mm,
        grid=(v // bm,),
        in_specs=[
            pl.BlockSpec(memory_space=pl.ANY),
            pl.BlockSpec((h, d), lambda i: (0, 0)),
            pl.BlockSpec((d, h), lambda i: (0, 0)),
        ],
        out_specs=pl.BlockSpec((bm, 2 * d), lambda i: (i, 0)),
        out_shape=jax.ShapeDtypeStruct((v, 2 * d), jnp.float32),
        scratch_shapes=[
            pltpu.VMEM((bm, d), jnp.float32),
            pltpu.SemaphoreType.DMA,
        ],
    )(table, w_down, w_up)


def _sc_gather_wide(table2, idx, chunk=256):
    n = idx.shape[0]
    w = table2.shape[1]
    n_per_w = n // NW
    n_chunks = n_per_w // chunk
    assert n_per_w * NW == n and n_chunks * chunk == n_per_w
    assert n_chunks % 2 == 0 and n_chunks >= 4

    mesh = plsc.VectorSubcoreMesh(
        core_axis_name="c", subcore_axis_name="s",
        num_cores=NC, num_subcores=NS)

    @functools.partial(
        pl.kernel, mesh=mesh,
        out_type=jax.ShapeDtypeStruct((n, w), jnp.float32),
        scratch_types=[
            pltpu.VMEM((n_per_w,), jnp.int32),
            pltpu.VMEM((2, chunk, w), jnp.float32),
            pltpu.SemaphoreType.DMA,
            pltpu.SemaphoreType.DMA,
            pltpu.SemaphoreType.DMA,
        ],
        compiler_params=pltpu.CompilerParams(use_tc_tiling_on_sc=True),
    )
    def gather_kernel(table_hbm, idx_hbm, out_hbm, idx_v, rows_v,
                      gsem0, gsem1, wsem):
        base = lax.axis_index("s") * NC + lax.axis_index("c")
        base = base * n_per_w
        pltpu.sync_copy(idx_hbm.at[pl.ds(base, n_per_w)], idx_v)

        def gather_grp(g, buf, sem):
            return pltpu.async_copy(
                table_hbm.at[idx_v.at[pl.ds(g * chunk, chunk)]],
                rows_v.at[buf], sem)

        def write_grp(g, buf):
            pltpu.async_copy(
                rows_v.at[buf], out_hbm.at[pl.ds(base + g * chunk, chunk)],
                wsem)

        def drain_write(g, buf):
            pltpu.make_async_copy(
                rows_v.at[buf], out_hbm.at[pl.ds(base + g * chunk, chunk)],
                wsem).wait()

        gather_grp(0, 0, gsem0)

        def body(g2, _):
            for sub in (0, 1):
                g = g2 * 2 + sub
                nbuf = (sub + 1) % 2
                nsem = gsem1 if sub == 0 else gsem0

                @pl.when(g + 1 < n_chunks)
                def _():
                    @pl.when(g >= 1)
                    def _():
                        drain_write(g - 1, nbuf)
                    gather_grp(g + 1, nbuf, nsem)

                pltpu.make_async_copy(
                    table_hbm.at[idx_v.at[pl.ds(g * chunk, chunk)]],
                    rows_v.at[sub], gsem0 if sub == 0 else gsem1).wait()
                write_grp(g, sub)
            return 0

        lax.fori_loop(0, n_chunks // 2, body, 0)
        drain_write(n_chunks - 2, 0)
        drain_write(n_chunks - 1, 1)

    return gather_kernel(table2, idx)


def kernel(x, table, W_down, W_up):
    b, l = x.shape
    d = table.shape[1]
    idx = x.reshape(b * l).astype(jnp.int32)
    table2 = _tc_transform(table, W_down, W_up)
    emb_wide = _sc_gather_wide(table2, idx)
    return emb_wide[:, :d].reshape(b, l, d)


# speedup vs baseline: 1.1898x; 1.1898x over previous
import functools

import jax
import jax.numpy as jnp
from jax import lax
from jax.experimental import pallas as pl
from jax.experimental.pallas import tpu as pltpu
from jax.experimental.pallas import tpu_sc as plsc

NC = 2
NS = 16
NW = NC * NS


def _tc_transform(table, w_down, w_up, bm=8000):
    v, d = table.shape
    h = w_down.shape[0]
    assert v % bm == 0

    n_blk = v // bm

    def mm(t_hbm, wd_ref, wu_ref, out_ref, t_vmem, sems):
        i = pl.program_id(0)

        def start_blk(j):
            pltpu.make_async_copy(
                t_hbm.at[pl.ds(j * bm, bm)],
                t_vmem.at[lax.rem(j, 2)], sems.at[lax.rem(j, 2)]).start()

        @pl.when(i == 0)
        def _():
            start_blk(i)

        @pl.when(i + 1 < n_blk)
        def _():
            start_blk(i + 1)

        wc = lax.dot_general(
            wd_ref[...], wu_ref[...], (((0,), (1,)), ((), ())),
            preferred_element_type=jnp.float32)
        pltpu.make_async_copy(
            t_hbm.at[pl.ds(i * bm, bm)],
            t_vmem.at[lax.rem(i, 2)], sems.at[lax.rem(i, 2)]).wait()
        res = lax.dot_general(
            t_vmem[lax.rem(i, 2)], wc, (((1,), (0,)), ((), ())),
            preferred_element_type=jnp.float32)
        out_ref[...] = jnp.concatenate([res, res], axis=1)

    return pl.pallas_call(
        mm,
        grid=(n_blk,),
        in_specs=[
            pl.BlockSpec(memory_space=pl.ANY),
            pl.BlockSpec((h, d), lambda i: (0, 0)),
            pl.BlockSpec((d, h), lambda i: (0, 0)),
        ],
        out_specs=pl.BlockSpec((bm, 2 * d), lambda i: (i, 0)),
        out_shape=jax.ShapeDtypeStruct((v, 2 * d), jnp.float32),
        scratch_shapes=[
            pltpu.VMEM((2, bm, d), jnp.float32),
            pltpu.SemaphoreType.DMA((2,)),
        ],
    )(table, w_down, w_up)


def _sc_gather_wide(table2, idx, chunk=256):
    n = idx.shape[0]
    w = table2.shape[1]
    n_per_w = n // NW
    n_chunks = n_per_w // chunk
    assert n_per_w * NW == n and n_chunks * chunk == n_per_w
    assert n_chunks % 2 == 0 and n_chunks >= 4

    mesh = plsc.VectorSubcoreMesh(
        core_axis_name="c", subcore_axis_name="s",
        num_cores=NC, num_subcores=NS)

    @functools.partial(
        pl.kernel, mesh=mesh,
        out_type=jax.ShapeDtypeStruct((n, w), jnp.float32),
        scratch_types=[
            pltpu.VMEM((n_per_w,), jnp.int32),
            pltpu.VMEM((2, chunk, w), jnp.float32),
            pltpu.SemaphoreType.DMA,
            pltpu.SemaphoreType.DMA,
            pltpu.SemaphoreType.DMA,
        ],
        compiler_params=pltpu.CompilerParams(use_tc_tiling_on_sc=True),
    )
    def gather_kernel(table_hbm, idx_hbm, out_hbm, idx_v, rows_v,
                      gsem0, gsem1, wsem):
        base = lax.axis_index("s") * NC + lax.axis_index("c")
        base = base * n_per_w
        pltpu.sync_copy(idx_hbm.at[pl.ds(base, n_per_w)], idx_v)

        def gather_grp(g, buf, sem):
            return pltpu.async_copy(
                table_hbm.at[idx_v.at[pl.ds(g * chunk, chunk)]],
                rows_v.at[buf], sem)

        def write_grp(g, buf):
            pltpu.async_copy(
                rows_v.at[buf], out_hbm.at[pl.ds(base + g * chunk, chunk)],
                wsem)

        def drain_write(g, buf):
            pltpu.make_async_copy(
                rows_v.at[buf], out_hbm.at[pl.ds(base + g * chunk, chunk)],
                wsem).wait()

        gather_grp(0, 0, gsem0)

        def body(g2, _):
            for sub in (0, 1):
                g = g2 * 2 + sub
                nbuf = (sub + 1) % 2
                nsem = gsem1 if sub == 0 else gsem0

                @pl.when(g + 1 < n_chunks)
                def _():
                    @pl.when(g >= 1)
                    def _():
                        drain_write(g - 1, nbuf)
                    gather_grp(g + 1, nbuf, nsem)

                pltpu.make_async_copy(
                    table_hbm.at[idx_v.at[pl.ds(g * chunk, chunk)]],
                    rows_v.at[sub], gsem0 if sub == 0 else gsem1).wait()
                write_grp(g, sub)
            return 0

        lax.fori_loop(0, n_chunks // 2, body, 0)
        drain_write(n_chunks - 2, 0)
        drain_write(n_chunks - 1, 1)

    return gather_kernel(table2, idx)


def kernel(x, table, W_down, W_up):
    b, l = x.shape
    d = table.shape[1]
    idx = x.reshape(b * l).astype(jnp.int32)
    table2 = _tc_transform(table, W_down, W_up)
    emb_wide = _sc_gather_wide(table2, idx)
    return emb_wide[:, :d].reshape(b, l, d)
